# baseline (speedup 1.0000x reference)
import jax
import jax.numpy as jnp
from jax import lax
from jax.experimental import pallas as pl
from jax.experimental.pallas import tpu as pltpu
from jax.experimental.pallas import tpu_sc as plsc

B, T, C = 4, 2048, 1024
TA = T // 2
TB = T // 2
R = 512
N_UNM = TA - R
HC = C // 2

_F32 = jnp.float32
_I32 = jnp.int32

_N_CORES = 2
_N_SUB = 16
_B_PER_CORE = B // _N_CORES
_UNM_PER_TILE = N_UNM // _N_SUB


def _select_body(k_ref, unm_glob_ref, dst_slot_ref, sm_ref):
    batch = pl.program_id(0)
    kv = k_ref[0]
    a = kv[:, :C]
    b = kv[:, C:]

    a_n = a / (jnp.sqrt(jnp.sum(a * a, axis=-1, keepdims=True)) + 1e-12)
    b_n = b / (jnp.sqrt(jnp.sum(b * b, axis=-1, keepdims=True)) + 1e-12)
    sim = lax.dot_general(
        a_n, b_n, (((1,), (1,)), ((), ())), preferred_element_type=_F32
    )

    ii = lax.broadcasted_iota(_I32, (TA, TB), 0)
    jj = lax.broadcasted_iota(_I32, (TA, TB), 1)
    neg_inf = jnp.float32(-jnp.inf)
    sim = jnp.where(ii == 0, neg_inf, sim)
    sim_t = jnp.transpose(sim)

    v_col = jnp.max(sim, axis=1, keepdims=True)
    v_row = jnp.max(sim_t, axis=0, keepdims=True)

    nidx_row = jnp.min(
        jnp.where(sim_t == v_row, ii, TB), axis=0, keepdims=True
    )

    ahead_c = (v_row > v_col) | ((v_row == v_col) & (jj < ii))
    rank_col = jnp.sum(ahead_c.astype(_I32), axis=1, keepdims=True)
    ahead_r = (v_col > v_row) | ((v_col == v_row) & (ii < jj))
    rank_row = jnp.sum(ahead_r.astype(_I32), axis=0, keepdims=True)

    src_row = rank_row < R
    unm_col = (rank_col >= R).astype(_I32)
    unm_pos_row = jnp.sum(
        jnp.where(ii < jj, unm_col, 0), axis=0, keepdims=True
    )
    unm_pos_col = jnp.sum(
        jnp.where((jj < ii) & (rank_row >= R), 1, 0), axis=1, keepdims=True
    )

    ii2 = lax.broadcasted_iota(_I32, (TA, R), 0)
    uu2 = lax.broadcasted_iota(_I32, (TA, R), 1)
    hit_unm = (unm_pos_col == uu2) & (rank_col >= R)
    inv_unm = jnp.sum(jnp.where(hit_unm, ii2, 0), axis=0, keepdims=True)

    unm_glob_ref[0] = batch * T + 2 * inv_unm
    dst_slot_ref[0] = jnp.where(src_row, nidx_row, -1)

    even_map = jnp.where(src_row, nidx_row + N_UNM, unm_pos_row)
    odd_map = lax.broadcasted_iota(_I32, (1, TB), 1) + N_UNM
    sm_ref[0, 0:1, :] = even_map
    sm_ref[0, 1:2, :] = odd_map


def _sc_gather_body(xf_hbm, unm_glob_hbm, unm_hbm, idx_v, rows_v):
    core = lax.axis_index("c")
    sub = lax.axis_index("s")
    u0 = sub * _UNM_PER_TILE
    for bb in range(_B_PER_CORE):
        bg = core * _B_PER_CORE + bb
        pltpu.sync_copy(unm_glob_hbm.at[bg, pl.ds(u0, _UNM_PER_TILE)], idx_v)
        pltpu.sync_copy(xf_hbm.at[idx_v], rows_v)
        pltpu.sync_copy(rows_v, unm_hbm.at[bg, pl.ds(u0, _UNM_PER_TILE)])


def _merge_body(xe_ref, xo_ref, unm_ref, dst_slot_ref, merged_ref):
    xe = xe_ref[0]
    xo = xo_ref[0]
    dst_slot = dst_slot_ref[0]

    d_iota = lax.broadcasted_iota(_I32, (TB, TA), 0)
    S = (dst_slot == d_iota).astype(_F32)
    dst_rows = xo + lax.dot_general(
        S, xe, (((1,), (0,)), ((), ())), preferred_element_type=_F32
    )

    merged_ref[0, :N_UNM, :] = unm_ref[0]
    merged_ref[0, N_UNM:, :] = dst_rows


@jax.jit
def kernel(x, k):
    k2 = k.reshape(B, TA, 2 * C)
    x2 = x.reshape(B, TA, 2 * C)
    xf = x.reshape(B * T, C)

    unm_glob, dst_slot, sm2 = pl.pallas_call(
        _select_body,
        grid=(B,),
        in_specs=[pl.BlockSpec((1, TA, 2 * C), lambda i: (i, 0, 0))],
        out_specs=[
            pl.BlockSpec((1, 1, R), lambda i: (i, 0, 0)),
            pl.BlockSpec((1, 1, TA), lambda i: (i, 0, 0)),
            pl.BlockSpec((1, 2, TA), lambda i: (i, 0, 0)),
        ],
        out_shape=[
            jax.ShapeDtypeStruct((B, 1, R), _I32),
            jax.ShapeDtypeStruct((B, 1, TA), _I32),
            jax.ShapeDtypeStruct((B, 2, TA), _I32),
        ],
    )(k2)

    mesh = plsc.VectorSubcoreMesh(core_axis_name="c", subcore_axis_name="s")
    sc_gather = pl.kernel(
        _sc_gather_body,
        mesh=mesh,
        out_type=jax.ShapeDtypeStruct((B, N_UNM, C), _F32),
        scratch_types=[
            pltpu.VMEM((_UNM_PER_TILE,), _I32),
            pltpu.VMEM((_UNM_PER_TILE, C), _F32),
        ],
    )
    unm = sc_gather(xf, unm_glob.reshape(B, R))

    merged = pl.pallas_call(
        _merge_body,
        grid=(B, 2),
        in_specs=[
            pl.BlockSpec((1, TA, HC), lambda i, h: (i, 0, h)),
            pl.BlockSpec((1, TA, HC), lambda i, h: (i, 0, 2 + h)),
            pl.BlockSpec((1, N_UNM, HC), lambda i, h: (i, 0, h)),
            pl.BlockSpec((1, 1, TA), lambda i, h: (i, 0, 0)),
        ],
        out_specs=pl.BlockSpec((1, N_UNM + TB, HC), lambda i, h: (i, 0, h)),
        out_shape=jax.ShapeDtypeStruct((B, N_UNM + TB, C), _F32),
    )(x2, x2, unm, dst_slot)

    source_map = jnp.transpose(sm2, (0, 2, 1)).reshape(B, T)
    return merged, source_map
